# baseline (speedup 1.0000x reference)
import jax
import jax.numpy as jnp
from jax import lax
from jax.experimental import pallas as pl
from jax.experimental.pallas import tpu as pltpu
from jax.experimental.pallas import tpu_sc as plsc

B = 16384
D = 60
DP = 64
NWIN = DP // 16
L = 20
NC = 2
NS = 16
NWKR = NC * NS
EPT = B // NWKR
G = 16
NCHUNK = EPT // G
NWORD = 100000
NREL = 1000
WPB = 125
WBLK = NWORD // NS // WPB
RPB = NREL // 8


def _rsqrt(a):
    i = plsc.bitcast(a, jnp.int32)
    i = 0x5F3759DF - lax.shift_right_arithmetic(i, 1)
    y = plsc.bitcast(i, jnp.float32)
    for _ in range(3):
        y = y * (1.5 - 0.5 * a * y * y)
    return y


def _sqrt(a):
    return a * _rsqrt(a)


def _body(head_hbm, rel_hbm, tail_hbm, hw_hbm, rw_hbm, tw_hbm,
          ent_hbm, rel_emb_hbm, word_hbm, proj_hbm,
          out_hbm, wpad, rpad, ppad,
          eh_idx, er_idx, et_idx, whw_idx, wrw_idx, wtw_idx,
          h_rows, t_rows, r_rows, p_rows, hw_rows, rw_rows, tw_rows,
          scores_v, wstage, fstage, sem):
    cid = lax.axis_index("c")
    sid = lax.axis_index("s")
    wid = sid * NC + cid
    base = wid * EPT

    iota = lax.iota(jnp.int32, 16)
    iota44 = iota + 44

    def pad_rows(src_hbm, dst_hbm, lo):
        pltpu.sync_copy(src_hbm.at[pl.ds(lo, WPB)], wstage)

        def rep(j, _):
            for u in range(5):
                r = j * 5 + u
                w0 = wstage[r, pl.ds(0, 16)]
                w1 = wstage[r, pl.ds(16, 16)]
                w2 = wstage[r, pl.ds(32, 16)]
                r_s = jnp.full((16,), 0, jnp.int32) + r
                w3 = plsc.load_gather(wstage, [r_s, iota44])
                fstage[r, pl.ds(0, 16)] = w0
                fstage[r, pl.ds(16, 16)] = w1
                fstage[r, pl.ds(32, 16)] = w2
                plsc.store_scatter(fstage, [r_s, iota44], w3)
            return 0

        lax.fori_loop(0, WPB // 5, rep, 0)
        pltpu.sync_copy(fstage, dst_hbm.at[cid, pl.ds(lo, WPB)])

    def pad_block(b, _):
        pad_rows(word_hbm, wpad, sid * (NWORD // NS) + b * WPB)
        return 0

    lax.fori_loop(0, WBLK, pad_block, 0)

    @pl.when(sid < 8)
    def _():
        pad_rows(rel_emb_hbm, rpad, sid * RPB)

    @pl.when(sid >= 8)
    def _():
        pad_rows(proj_hbm, ppad, (sid - 8) * RPB)

    plsc.subcore_barrier()

    pltpu.sync_copy(head_hbm.at[pl.ds(base, EPT)], eh_idx)
    pltpu.sync_copy(rel_hbm.at[pl.ds(base, EPT)], er_idx)
    pltpu.sync_copy(tail_hbm.at[pl.ds(base, EPT)], et_idx)
    pltpu.sync_copy(hw_hbm.at[pl.ds(base * L, EPT * L)], whw_idx)
    pltpu.sync_copy(rw_hbm.at[pl.ds(base * L, EPT * L)], wrw_idx)
    pltpu.sync_copy(tw_hbm.at[pl.ds(base * L, EPT * L)], wtw_idx)

    inv_l = jnp.float32(1.0 / L)
    zero16 = jnp.zeros((16,), jnp.float32)
    iota_c12 = jnp.minimum(iota, 11) + 48
    m12 = jnp.where(iota < 12, jnp.float32(1.0), jnp.float32(0.0))

    wpad_c = wpad.at[cid]
    rpad_c = rpad.at[cid]
    ppad_c = ppad.at[cid]

    def chunk_body(c, _):
        eb = c * G
        descs = []
        e_h = eh_idx[pl.ds(eb, G)]
        e_t = et_idx[pl.ds(eb, G)]
        for k in range(G):
            descs.append(pltpu.async_copy(
                ent_hbm.at[pl.ds(e_h[k], 1)], h_rows.at[pl.ds(k, 1)], sem))
            descs.append(pltpu.async_copy(
                ent_hbm.at[pl.ds(e_t[k], 1)], t_rows.at[pl.ds(k, 1)], sem))
        descs.append(pltpu.async_copy(
            rpad_c.at[er_idx.at[pl.ds(eb, G)]], r_rows, sem))
        descs.append(pltpu.async_copy(
            ppad_c.at[er_idx.at[pl.ds(eb, G)]], p_rows, sem))
        for j in range(4):
            o = j * 80
            descs.append(pltpu.async_copy(
                wpad_c.at[whw_idx.at[pl.ds(eb * L + o, 80)]],
                hw_rows.at[pl.ds(o, 80)], sem))
            descs.append(pltpu.async_copy(
                wpad_c.at[wrw_idx.at[pl.ds(eb * L + o, 80)]],
                rw_rows.at[pl.ds(o, 80)], sem))
            descs.append(pltpu.async_copy(
                wpad_c.at[wtw_idx.at[pl.ds(eb * L + o, 80)]],
                tw_rows.at[pl.ds(o, 80)], sem))
        for dsc in descs:
            dsc.wait()

        def elem_body(i, ss_acc):
            rb = i * L
            i_s = jnp.full((16,), 0, jnp.int32) + i
            hv = [h_rows[i, pl.ds(w * 16, 16)] for w in range(NWIN - 1)]
            hv.append(plsc.load_gather(h_rows, [i_s, iota_c12]) * m12)
            tv = [t_rows[i, pl.ds(w * 16, 16)] for w in range(NWIN - 1)]
            tv.append(plsc.load_gather(t_rows, [i_s, iota_c12]) * m12)
            rv = [r_rows[i, pl.ds(w * 16, 16)] for w in range(NWIN)]
            pv = [p_rows[i, pl.ds(w * 16, 16)] for w in range(NWIN)]
            pv[3] = pv[3] * m12
            hs = [zero16] * NWIN
            rs = [zero16] * NWIN
            ts = [zero16] * NWIN
            for l in range(L):
                r = rb + l
                for w in range(NWIN):
                    o = w * 16
                    hs[w] = hs[w] + hw_rows[r, pl.ds(o, 16)]
                    rs[w] = rs[w] + rw_rows[r, pl.ds(o, 16)]
                    ts[w] = ts[w] + tw_rows[r, pl.ds(o, 16)]
            he = [hv[w] + hs[w] * inv_l for w in range(NWIN)]
            re = [rv[w] + rs[w] * inv_l for w in range(NWIN)]
            te = [tv[w] + ts[w] * inv_l for w in range(NWIN)]
            he[3] = he[3] * m12
            re[3] = re[3] * m12
            te[3] = te[3] * m12
            ppv = pv[0] * pv[0]
            phv = pv[0] * he[0]
            ptv = pv[0] * te[0]
            for w in range(1, NWIN):
                ppv = ppv + pv[w] * pv[w]
                phv = phv + pv[w] * he[w]
                ptv = ptv + pv[w] * te[w]
            pp = jnp.sum(ppv)
            ph = jnp.sum(phv)
            pt = jnp.sum(ptv)
            pp_s = jnp.zeros((16,), jnp.float32) + pp
            m = jnp.maximum(_sqrt(pp_s), jnp.float32(1e-12))
            cv = (jnp.zeros((16,), jnp.float32) + (ph - pt)) / (m * m)
            ssv = zero16
            for w in range(NWIN):
                v = he[w] + re[w] - te[w] - cv * pv[w]
                ssv = ssv + v * v
            ss = jnp.sum(ssv)
            return jnp.where(iota == i, jnp.zeros((16,), jnp.float32) + ss,
                             ss_acc)

        ss_acc = lax.fori_loop(0, G, elem_body, zero16)
        scores_v[...] = -_sqrt(ss_acc)
        pltpu.sync_copy(scores_v, out_hbm.at[pl.ds(base + eb, G)])
        return 0

    lax.fori_loop(0, NCHUNK, chunk_body, 0)


SCRATCH = [
    pltpu.VMEM((EPT,), jnp.int32),
    pltpu.VMEM((EPT,), jnp.int32),
    pltpu.VMEM((EPT,), jnp.int32),
    pltpu.VMEM((EPT * L,), jnp.int32),
    pltpu.VMEM((EPT * L,), jnp.int32),
    pltpu.VMEM((EPT * L,), jnp.int32),
    pltpu.VMEM((G, D), jnp.float32),
    pltpu.VMEM((G, D), jnp.float32),
    pltpu.VMEM((G, DP), jnp.float32),
    pltpu.VMEM((G, DP), jnp.float32),
    pltpu.VMEM((G * L, DP), jnp.float32),
    pltpu.VMEM((G * L, DP), jnp.float32),
    pltpu.VMEM((G * L, DP), jnp.float32),
    pltpu.VMEM((G,), jnp.float32),
    pltpu.VMEM((WPB, D), jnp.float32),
    pltpu.VMEM((WPB, DP), jnp.float32),
    pltpu.SemaphoreType.DMA,
]


@jax.jit
def _transh_sc(head, relation, tail, hw_flat, rw_flat, tw_flat,
               entity_embedding, relation_embedding, word_embedding,
               relation_projection):
    mesh = plsc.VectorSubcoreMesh(core_axis_name="c", subcore_axis_name="s")
    f = pl.kernel(
        _body,
        out_type=(
            jax.ShapeDtypeStruct((B,), jnp.float32),
            jax.ShapeDtypeStruct((NC, NWORD, DP), jnp.float32),
            jax.ShapeDtypeStruct((NC, NREL, DP), jnp.float32),
            jax.ShapeDtypeStruct((NC, NREL, DP), jnp.float32),
        ),
        mesh=mesh,
        compiler_params=pltpu.CompilerParams(
            needs_layout_passes=False, use_tc_tiling_on_sc=False),
        scratch_types=SCRATCH,
    )
    score, _, _, _ = f(head, relation, tail, hw_flat, rw_flat, tw_flat,
                       entity_embedding, relation_embedding, word_embedding,
                       relation_projection)
    return score


def kernel(head, relation, tail, head_w, rel_w, tail_w,
           entity_embedding, relation_embedding, word_embedding,
           relation_projection):
    i32 = jnp.int32
    return _transh_sc(
        head.astype(i32), relation.astype(i32), tail.astype(i32),
        head_w.astype(i32).reshape(-1), rel_w.astype(i32).reshape(-1),
        tail_w.astype(i32).reshape(-1),
        entity_embedding, relation_embedding, word_embedding,
        relation_projection)


# speedup vs baseline: 1.8563x; 1.8563x over previous
import jax
import jax.numpy as jnp
from jax import lax
from jax.experimental import pallas as pl
from jax.experimental.pallas import tpu as pltpu
from jax.experimental.pallas import tpu_sc as plsc

B = 16384
D = 60
DP = 64
NWIN = DP // 16
L = 20
NC = 2
NS = 16
NWKR = NC * NS
EPT = B // NWKR
G = 16
NCHUNK = EPT // G


def _rsqrt(a):
    i = plsc.bitcast(a, jnp.int32)
    i = 0x5F3759DF - lax.shift_right_arithmetic(i, 1)
    y = plsc.bitcast(i, jnp.float32)
    for _ in range(3):
        y = y * (1.5 - 0.5 * a * y * y)
    return y


def _sqrt(a):
    return a * _rsqrt(a)


def _gath_body(ent_hbm, head_hbm, tail_hbm, out_hbm, idx_v, rows_v, sem):
    wid = lax.axis_index("s") * NC + lax.axis_index("c")
    base = wid * EPT

    def do_half(src_idx, out_base):
        pltpu.sync_copy(src_idx.at[pl.ds(base, EPT)], idx_v)

        def blk(b, _):
            e = idx_v[pl.ds(b * G, G)]
            descs = [
                pltpu.async_copy(ent_hbm.at[pl.ds(e[k], 1)],
                                 rows_v.at[pl.ds(k, 1)], sem)
                for k in range(G)
            ]
            for dsc in descs:
                dsc.wait()
            pltpu.sync_copy(rows_v, out_hbm.at[pl.ds(out_base + b * G, G)])
            return 0

        lax.fori_loop(0, NCHUNK, blk, 0)

    do_half(head_hbm, base)
    do_half(tail_hbm, B + base)


def _body(head_hbm, rel_hbm, tail_hbm, hw_hbm, rw_hbm, tw_hbm,
          ent_rows_hbm, rel_emb_hbm, word_hbm, proj_hbm, out_hbm,
          er_idx, whw_idx, wrw_idx, wtw_idx,
          h_rows, t_rows, r_rows, p_rows, hw_rows, rw_rows, tw_rows,
          scores_v, sem):
    wid = lax.axis_index("s") * NC + lax.axis_index("c")
    base = wid * EPT

    pltpu.sync_copy(rel_hbm.at[pl.ds(base, EPT)], er_idx)
    pltpu.sync_copy(hw_hbm.at[pl.ds(base * L, EPT * L)], whw_idx)
    pltpu.sync_copy(rw_hbm.at[pl.ds(base * L, EPT * L)], wrw_idx)
    pltpu.sync_copy(tw_hbm.at[pl.ds(base * L, EPT * L)], wtw_idx)

    iota = lax.iota(jnp.int32, 16)
    inv_l = jnp.float32(1.0 / L)
    zero16 = jnp.zeros((16,), jnp.float32)
    iota_c12 = jnp.minimum(iota, 11) + 48
    m12 = jnp.where(iota < 12, jnp.float32(1.0), jnp.float32(0.0))

    def chunk_body(c, _):
        eb = c * G
        descs = [
            pltpu.async_copy(ent_rows_hbm.at[pl.ds(base + eb, G)],
                             h_rows, sem),
            pltpu.async_copy(ent_rows_hbm.at[pl.ds(B + base + eb, G)],
                             t_rows, sem),
            pltpu.async_copy(rel_emb_hbm.at[er_idx.at[pl.ds(eb, G)]],
                             r_rows, sem),
            pltpu.async_copy(proj_hbm.at[er_idx.at[pl.ds(eb, G)]],
                             p_rows, sem),
        ]
        for j in range(4):
            o = j * 80
            descs.append(pltpu.async_copy(
                word_hbm.at[whw_idx.at[pl.ds(eb * L + o, 80)]],
                hw_rows.at[pl.ds(o, 80)], sem))
            descs.append(pltpu.async_copy(
                word_hbm.at[wrw_idx.at[pl.ds(eb * L + o, 80)]],
                rw_rows.at[pl.ds(o, 80)], sem))
            descs.append(pltpu.async_copy(
                word_hbm.at[wtw_idx.at[pl.ds(eb * L + o, 80)]],
                tw_rows.at[pl.ds(o, 80)], sem))
        for dsc in descs:
            dsc.wait()

        def elem_body(i, ss_acc):
            rb = i * L
            i_s = jnp.full((16,), 0, jnp.int32) + i
            hv = [h_rows[i, pl.ds(w * 16, 16)] for w in range(NWIN - 1)]
            hv.append(plsc.load_gather(h_rows, [i_s, iota_c12]) * m12)
            tv = [t_rows[i, pl.ds(w * 16, 16)] for w in range(NWIN - 1)]
            tv.append(plsc.load_gather(t_rows, [i_s, iota_c12]) * m12)
            rv = [r_rows[i, pl.ds(w * 16, 16)] for w in range(NWIN)]
            pv = [p_rows[i, pl.ds(w * 16, 16)] for w in range(NWIN)]
            hs = [zero16] * NWIN
            rs = [zero16] * NWIN
            ts = [zero16] * NWIN
            for l in range(L):
                r = rb + l
                for w in range(NWIN):
                    o = w * 16
                    hs[w] = hs[w] + hw_rows[r, pl.ds(o, 16)]
                    rs[w] = rs[w] + rw_rows[r, pl.ds(o, 16)]
                    ts[w] = ts[w] + tw_rows[r, pl.ds(o, 16)]
            he = [hv[w] + hs[w] * inv_l for w in range(NWIN)]
            re = [rv[w] + rs[w] * inv_l for w in range(NWIN)]
            te = [tv[w] + ts[w] * inv_l for w in range(NWIN)]
            ppv = pv[0] * pv[0]
            phv = pv[0] * he[0]
            ptv = pv[0] * te[0]
            for w in range(1, NWIN):
                ppv = ppv + pv[w] * pv[w]
                phv = phv + pv[w] * he[w]
                ptv = ptv + pv[w] * te[w]
            pp = jnp.sum(ppv)
            ph = jnp.sum(phv)
            pt = jnp.sum(ptv)
            pp_s = jnp.zeros((16,), jnp.float32) + pp
            m = jnp.maximum(_sqrt(pp_s), jnp.float32(1e-12))
            cv = (jnp.zeros((16,), jnp.float32) + (ph - pt)) / (m * m)
            ssv = zero16
            for w in range(NWIN):
                v = he[w] + re[w] - te[w] - cv * pv[w]
                ssv = ssv + v * v
            ss = jnp.sum(ssv)
            return jnp.where(iota == i, jnp.zeros((16,), jnp.float32) + ss,
                             ss_acc)

        ss_acc = lax.fori_loop(0, G, elem_body, zero16)
        scores_v[...] = -_sqrt(ss_acc)
        pltpu.sync_copy(scores_v, out_hbm.at[pl.ds(base + eb, G)])
        return 0

    lax.fori_loop(0, NCHUNK, chunk_body, 0)


SCRATCH = [
    pltpu.VMEM((EPT,), jnp.int32),
    pltpu.VMEM((EPT * L,), jnp.int32),
    pltpu.VMEM((EPT * L,), jnp.int32),
    pltpu.VMEM((EPT * L,), jnp.int32),
    pltpu.VMEM((G, D), jnp.float32),
    pltpu.VMEM((G, D), jnp.float32),
    pltpu.VMEM((G, DP), jnp.float32),
    pltpu.VMEM((G, DP), jnp.float32),
    pltpu.VMEM((G * L, DP), jnp.float32),
    pltpu.VMEM((G * L, DP), jnp.float32),
    pltpu.VMEM((G * L, DP), jnp.float32),
    pltpu.VMEM((G,), jnp.float32),
    pltpu.SemaphoreType.DMA,
]


@jax.jit
def _transh_sc(head, relation, tail, hw_flat, rw_flat, tw_flat,
               entity_embedding, rel_emb_p, word_p, proj_p):
    mesh = plsc.VectorSubcoreMesh(core_axis_name="c", subcore_axis_name="s")
    gather_ent = pl.kernel(
        _gath_body,
        out_type=jax.ShapeDtypeStruct((2 * B, D), jnp.float32),
        mesh=mesh,
        compiler_params=pltpu.CompilerParams(
            needs_layout_passes=False, use_tc_tiling_on_sc=True),
        scratch_types=[
            pltpu.VMEM((EPT,), jnp.int32),
            pltpu.VMEM((G, D), jnp.float32),
            pltpu.SemaphoreType.DMA,
        ],
    )
    ent_rows = gather_ent(entity_embedding, head, tail)
    f = pl.kernel(
        _body,
        out_type=jax.ShapeDtypeStruct((B,), jnp.float32),
        mesh=mesh,
        compiler_params=pltpu.CompilerParams(
            needs_layout_passes=False, use_tc_tiling_on_sc=False),
        scratch_types=SCRATCH,
    )
    return f(head, relation, tail, hw_flat, rw_flat, tw_flat,
             ent_rows, rel_emb_p, word_p, proj_p)


def kernel(head, relation, tail, head_w, rel_w, tail_w,
           entity_embedding, relation_embedding, word_embedding,
           relation_projection):
    i32 = jnp.int32
    pad = ((0, 0), (0, DP - D))
    return _transh_sc(
        head.astype(i32), relation.astype(i32), tail.astype(i32),
        head_w.astype(i32).reshape(-1), rel_w.astype(i32).reshape(-1),
        tail_w.astype(i32).reshape(-1),
        entity_embedding,
        jnp.pad(relation_embedding, pad),
        jnp.pad(word_embedding, pad),
        jnp.pad(relation_projection, pad))


# speedup vs baseline: 2.1198x; 1.1420x over previous
import jax
import jax.numpy as jnp
from jax import lax
from jax.experimental import pallas as pl
from jax.experimental.pallas import tpu as pltpu
from jax.experimental.pallas import tpu_sc as plsc

B = 16384
D = 60
DP = 64
NWIN = DP // 16
L = 20
NC = 2
NS = 16
NWKR = NC * NS
EPT = B // NWKR
G = 8
NCHUNK = EPT // G


def _rsqrt(a):
    i = plsc.bitcast(a, jnp.int32)
    i = 0x5F3759DF - lax.shift_right_arithmetic(i, 1)
    y = plsc.bitcast(i, jnp.float32)
    for _ in range(3):
        y = y * (1.5 - 0.5 * a * y * y)
    return y


def _sqrt(a):
    return a * _rsqrt(a)


def _gath_body(ent_hbm, head_hbm, tail_hbm, out_hbm, idx_v, rows_v, sem):
    wid = lax.axis_index("s") * NC + lax.axis_index("c")
    base = wid * EPT

    GB = 16

    def do_half(src_idx, out_base):
        pltpu.sync_copy(src_idx.at[pl.ds(base, EPT)], idx_v)

        def blk(b, _):
            e = idx_v[pl.ds(b * GB, GB)]
            descs = [
                pltpu.async_copy(ent_hbm.at[pl.ds(e[k], 1)],
                                 rows_v.at[pl.ds(k, 1)], sem)
                for k in range(GB)
            ]
            for dsc in descs:
                dsc.wait()
            pltpu.sync_copy(rows_v, out_hbm.at[pl.ds(out_base + b * GB, GB)])
            return 0

        lax.fori_loop(0, EPT // GB, blk, 0)

    do_half(head_hbm, base)
    do_half(tail_hbm, B + base)


def _body(head_hbm, rel_hbm, tail_hbm, hw_hbm, rw_hbm, tw_hbm,
          ent_rows_hbm, rel_emb_hbm, word_hbm, proj_hbm, out_hbm,
          er_idx, whw_idx, wrw_idx, wtw_idx,
          h_rowsA, t_rowsA, r_rowsA, p_rowsA, hw_rowsA, rw_rowsA, tw_rowsA,
          h_rowsB, t_rowsB, r_rowsB, p_rowsB, hw_rowsB, rw_rowsB, tw_rowsB,
          scores_v, semA, semB):
    wid = lax.axis_index("s") * NC + lax.axis_index("c")
    base = wid * EPT

    pltpu.sync_copy(rel_hbm.at[pl.ds(base, EPT)], er_idx)
    pltpu.sync_copy(hw_hbm.at[pl.ds(base * L, EPT * L)], whw_idx)
    pltpu.sync_copy(rw_hbm.at[pl.ds(base * L, EPT * L)], wrw_idx)
    pltpu.sync_copy(tw_hbm.at[pl.ds(base * L, EPT * L)], wtw_idx)

    iota = lax.iota(jnp.int32, 16)
    inv_l = jnp.float32(1.0 / L)
    zero16 = jnp.zeros((16,), jnp.float32)
    iota_c12 = jnp.minimum(iota, 11) + 48
    m12 = jnp.where(iota < 12, jnp.float32(1.0), jnp.float32(0.0))

    bufA = (h_rowsA, t_rowsA, r_rowsA, p_rowsA, hw_rowsA, rw_rowsA, tw_rowsA)
    bufB = (h_rowsB, t_rowsB, r_rowsB, p_rowsB, hw_rowsB, rw_rowsB, tw_rowsB)

    def copies(c, bf):
        h_rows, t_rows, r_rows, p_rows, hw_rows, rw_rows, tw_rows = bf
        eb = c * G
        out = [
            (ent_rows_hbm.at[pl.ds(base + eb, G)], h_rows),
            (ent_rows_hbm.at[pl.ds(B + base + eb, G)], t_rows),
            (rel_emb_hbm.at[er_idx.at[pl.ds(eb, G)]], r_rows),
            (proj_hbm.at[er_idx.at[pl.ds(eb, G)]], p_rows),
        ]
        for j in range(2):
            o = j * 80
            out.append((word_hbm.at[whw_idx.at[pl.ds(eb * L + o, 80)]],
                        hw_rows.at[pl.ds(o, 80)]))
            out.append((word_hbm.at[wrw_idx.at[pl.ds(eb * L + o, 80)]],
                        rw_rows.at[pl.ds(o, 80)]))
            out.append((word_hbm.at[wtw_idx.at[pl.ds(eb * L + o, 80)]],
                        tw_rows.at[pl.ds(o, 80)]))
        return out

    def fire(c, bf, sem):
        for src, dst in copies(c, bf):
            pltpu.async_copy(src, dst, sem)

    def drain(c, bf, sem):
        for src, dst in copies(c, bf):
            pltpu.make_async_copy(src, dst, sem).wait()

    def compute(c, bf):
        h_rows, t_rows, r_rows, p_rows, hw_rows, rw_rows, tw_rows = bf
        eb = c * G

        def elem_body(i, ss_acc):
            rb = i * L
            i_s = jnp.full((16,), 0, jnp.int32) + i
            hv = [h_rows[i, pl.ds(w * 16, 16)] for w in range(NWIN - 1)]
            hv.append(plsc.load_gather(h_rows, [i_s, iota_c12]) * m12)
            tv = [t_rows[i, pl.ds(w * 16, 16)] for w in range(NWIN - 1)]
            tv.append(plsc.load_gather(t_rows, [i_s, iota_c12]) * m12)
            rv = [r_rows[i, pl.ds(w * 16, 16)] for w in range(NWIN)]
            pv = [p_rows[i, pl.ds(w * 16, 16)] for w in range(NWIN)]
            hs = [zero16] * NWIN
            rs = [zero16] * NWIN
            ts = [zero16] * NWIN
            for l in range(L):
                r = rb + l
                for w in range(NWIN):
                    o = w * 16
                    hs[w] = hs[w] + hw_rows[r, pl.ds(o, 16)]
                    rs[w] = rs[w] + rw_rows[r, pl.ds(o, 16)]
                    ts[w] = ts[w] + tw_rows[r, pl.ds(o, 16)]
            he = [hv[w] + hs[w] * inv_l for w in range(NWIN)]
            re = [rv[w] + rs[w] * inv_l for w in range(NWIN)]
            te = [tv[w] + ts[w] * inv_l for w in range(NWIN)]
            ppv = pv[0] * pv[0]
            phv = pv[0] * he[0]
            ptv = pv[0] * te[0]
            for w in range(1, NWIN):
                ppv = ppv + pv[w] * pv[w]
                phv = phv + pv[w] * he[w]
                ptv = ptv + pv[w] * te[w]
            pp = jnp.sum(ppv)
            ph = jnp.sum(phv)
            pt = jnp.sum(ptv)
            pp_s = jnp.zeros((16,), jnp.float32) + pp
            m = jnp.maximum(_sqrt(pp_s), jnp.float32(1e-12))
            cv = (jnp.zeros((16,), jnp.float32) + (ph - pt)) / (m * m)
            ssv = zero16
            for w in range(NWIN):
                v = he[w] + re[w] - te[w] - cv * pv[w]
                ssv = ssv + v * v
            ss = jnp.sum(ssv)
            return jnp.where(iota == i, jnp.zeros((16,), jnp.float32) + ss,
                             ss_acc)

        ss_acc = lax.fori_loop(0, G, elem_body, zero16)
        scores_v[...] = -_sqrt(ss_acc)
        pltpu.sync_copy(scores_v.at[pl.ds(0, G)],
                        out_hbm.at[pl.ds(base + eb, G)])

    fire(0, bufA, semA)

    def pair_body(c2, _):
        c = c2 * 2
        fire(c + 1, bufB, semB)
        drain(c, bufA, semA)
        compute(c, bufA)

        @pl.when(c + 2 < NCHUNK)
        def _():
            fire(c + 2, bufA, semA)

        drain(c + 1, bufB, semB)
        compute(c + 1, bufB)
        return 0

    lax.fori_loop(0, NCHUNK // 2, pair_body, 0)


_BUF = [
    pltpu.VMEM((G, D), jnp.float32),
    pltpu.VMEM((G, D), jnp.float32),
    pltpu.VMEM((G, DP), jnp.float32),
    pltpu.VMEM((G, DP), jnp.float32),
    pltpu.VMEM((G * L, DP), jnp.float32),
    pltpu.VMEM((G * L, DP), jnp.float32),
    pltpu.VMEM((G * L, DP), jnp.float32),
]

SCRATCH = [
    pltpu.VMEM((EPT,), jnp.int32),
    pltpu.VMEM((EPT * L,), jnp.int32),
    pltpu.VMEM((EPT * L,), jnp.int32),
    pltpu.VMEM((EPT * L,), jnp.int32),
    *_BUF,
    *_BUF,
    pltpu.VMEM((16,), jnp.float32),
    pltpu.SemaphoreType.DMA,
    pltpu.SemaphoreType.DMA,
]


@jax.jit
def _transh_sc(head, relation, tail, hw_flat, rw_flat, tw_flat,
               entity_embedding, rel_emb_p, word_p, proj_p):
    mesh = plsc.VectorSubcoreMesh(core_axis_name="c", subcore_axis_name="s")
    gather_ent = pl.kernel(
        _gath_body,
        out_type=jax.ShapeDtypeStruct((2 * B, D), jnp.float32),
        mesh=mesh,
        compiler_params=pltpu.CompilerParams(
            needs_layout_passes=False, use_tc_tiling_on_sc=True),
        scratch_types=[
            pltpu.VMEM((EPT,), jnp.int32),
            pltpu.VMEM((16, D), jnp.float32),
            pltpu.SemaphoreType.DMA,
        ],
    )
    ent_rows = gather_ent(entity_embedding, head, tail)
    f = pl.kernel(
        _body,
        out_type=jax.ShapeDtypeStruct((B,), jnp.float32),
        mesh=mesh,
        compiler_params=pltpu.CompilerParams(
            needs_layout_passes=False, use_tc_tiling_on_sc=False),
        scratch_types=SCRATCH,
    )
    return f(head, relation, tail, hw_flat, rw_flat, tw_flat,
             ent_rows, rel_emb_p, word_p, proj_p)


def kernel(head, relation, tail, head_w, rel_w, tail_w,
           entity_embedding, relation_embedding, word_embedding,
           relation_projection):
    i32 = jnp.int32
    pad = ((0, 0), (0, DP - D))
    return _transh_sc(
        head.astype(i32), relation.astype(i32), tail.astype(i32),
        head_w.astype(i32).reshape(-1), rel_w.astype(i32).reshape(-1),
        tail_w.astype(i32).reshape(-1),
        entity_embedding,
        jnp.pad(relation_embedding, pad),
        jnp.pad(word_embedding, pad),
        jnp.pad(relation_projection, pad))


# speedup vs baseline: 2.1225x; 1.0013x over previous
import jax
import jax.numpy as jnp
from jax import lax
from jax.experimental import pallas as pl
from jax.experimental.pallas import tpu as pltpu
from jax.experimental.pallas import tpu_sc as plsc

B = 16384
D = 60
DP = 64
NWIN = DP // 16
L = 20
NC = 2
NS = 16
NWKR = NC * NS
EPT = B // NWKR
G = 8
NCHUNK = EPT // G


def _rsqrt(a):
    i = plsc.bitcast(a, jnp.int32)
    i = 0x5F3759DF - lax.shift_right_arithmetic(i, 1)
    y = plsc.bitcast(i, jnp.float32)
    for _ in range(3):
        y = y * (1.5 - 0.5 * a * y * y)
    return y


def _sqrt(a):
    return a * _rsqrt(a)


def _gath_body(ent_hbm, head_hbm, tail_hbm, out_hbm,
               idx_v, rows_vA, rows_vB, semA, semB):
    wid = lax.axis_index("s") * NC + lax.axis_index("c")
    base = wid * EPT

    GB = 16
    NB = EPT // GB

    def do_half(src_idx, out_base):
        pltpu.sync_copy(src_idx.at[pl.ds(base, EPT)], idx_v)

        def fire(b, rows_v, sem):
            e = idx_v[pl.ds(b * GB, GB)]
            for k in range(GB):
                pltpu.async_copy(ent_hbm.at[pl.ds(e[k], 1)],
                                 rows_v.at[pl.ds(k, 1)], sem)

        def drain_write(b, rows_v, sem):
            e = idx_v[pl.ds(b * GB, GB)]
            for k in range(GB):
                pltpu.make_async_copy(ent_hbm.at[pl.ds(e[k], 1)],
                                      rows_v.at[pl.ds(k, 1)], sem).wait()
            pltpu.sync_copy(rows_v, out_hbm.at[pl.ds(out_base + b * GB, GB)])

        fire(0, rows_vA, semA)

        def pair(b2, _):
            b = b2 * 2
            fire(b + 1, rows_vB, semB)
            drain_write(b, rows_vA, semA)

            @pl.when(b + 2 < NB)
            def _():
                fire(b + 2, rows_vA, semA)

            drain_write(b + 1, rows_vB, semB)
            return 0

        lax.fori_loop(0, NB // 2, pair, 0)

    do_half(head_hbm, base)
    do_half(tail_hbm, B + base)


def _body(head_hbm, rel_hbm, tail_hbm, hw_hbm, rw_hbm, tw_hbm,
          ent_rows_hbm, rel_emb_hbm, word_hbm, proj_hbm, out_hbm,
          er_idx, whw_idx, wrw_idx, wtw_idx,
          h_rowsA, t_rowsA, r_rowsA, p_rowsA, hw_rowsA, rw_rowsA, tw_rowsA,
          h_rowsB, t_rowsB, r_rowsB, p_rowsB, hw_rowsB, rw_rowsB, tw_rowsB,
          scores_v, semA, semB):
    wid = lax.axis_index("s") * NC + lax.axis_index("c")
    base = wid * EPT

    pltpu.sync_copy(rel_hbm.at[pl.ds(base, EPT)], er_idx)
    pltpu.sync_copy(hw_hbm.at[pl.ds(base * L, EPT * L)], whw_idx)
    pltpu.sync_copy(rw_hbm.at[pl.ds(base * L, EPT * L)], wrw_idx)
    pltpu.sync_copy(tw_hbm.at[pl.ds(base * L, EPT * L)], wtw_idx)

    iota = lax.iota(jnp.int32, 16)
    inv_l = jnp.float32(1.0 / L)
    zero16 = jnp.zeros((16,), jnp.float32)
    iota_c12 = jnp.minimum(iota, 11) + 48
    m12 = jnp.where(iota < 12, jnp.float32(1.0), jnp.float32(0.0))

    bufA = (h_rowsA, t_rowsA, r_rowsA, p_rowsA, hw_rowsA, rw_rowsA, tw_rowsA)
    bufB = (h_rowsB, t_rowsB, r_rowsB, p_rowsB, hw_rowsB, rw_rowsB, tw_rowsB)

    def copies(c, bf):
        h_rows, t_rows, r_rows, p_rows, hw_rows, rw_rows, tw_rows = bf
        eb = c * G
        out = [
            (ent_rows_hbm.at[pl.ds(base + eb, G)], h_rows),
            (ent_rows_hbm.at[pl.ds(B + base + eb, G)], t_rows),
            (rel_emb_hbm.at[er_idx.at[pl.ds(eb, G)]], r_rows),
            (proj_hbm.at[er_idx.at[pl.ds(eb, G)]], p_rows),
        ]
        for j in range(2):
            o = j * 80
            out.append((word_hbm.at[whw_idx.at[pl.ds(eb * L + o, 80)]],
                        hw_rows.at[pl.ds(o, 80)]))
            out.append((word_hbm.at[wrw_idx.at[pl.ds(eb * L + o, 80)]],
                        rw_rows.at[pl.ds(o, 80)]))
            out.append((word_hbm.at[wtw_idx.at[pl.ds(eb * L + o, 80)]],
                        tw_rows.at[pl.ds(o, 80)]))
        return out

    def fire(c, bf, sem):
        for src, dst in copies(c, bf):
            pltpu.async_copy(src, dst, sem)

    def drain(c, bf, sem):
        for src, dst in copies(c, bf):
            pltpu.make_async_copy(src, dst, sem).wait()

    def compute(c, bf):
        h_rows, t_rows, r_rows, p_rows, hw_rows, rw_rows, tw_rows = bf
        eb = c * G

        def elem_body(i, ss_acc):
            rb = i * L
            i_s = jnp.full((16,), 0, jnp.int32) + i
            hv = [h_rows[i, pl.ds(w * 16, 16)] for w in range(NWIN - 1)]
            hv.append(plsc.load_gather(h_rows, [i_s, iota_c12]) * m12)
            tv = [t_rows[i, pl.ds(w * 16, 16)] for w in range(NWIN - 1)]
            tv.append(plsc.load_gather(t_rows, [i_s, iota_c12]) * m12)
            rv = [r_rows[i, pl.ds(w * 16, 16)] for w in range(NWIN)]
            pv = [p_rows[i, pl.ds(w * 16, 16)] for w in range(NWIN)]
            hs = [zero16] * NWIN
            rs = [zero16] * NWIN
            ts = [zero16] * NWIN
            for l in range(L):
                r = rb + l
                for w in range(NWIN):
                    o = w * 16
                    hs[w] = hs[w] + hw_rows[r, pl.ds(o, 16)]
                    rs[w] = rs[w] + rw_rows[r, pl.ds(o, 16)]
                    ts[w] = ts[w] + tw_rows[r, pl.ds(o, 16)]
            he = [hv[w] + hs[w] * inv_l for w in range(NWIN)]
            re = [rv[w] + rs[w] * inv_l for w in range(NWIN)]
            te = [tv[w] + ts[w] * inv_l for w in range(NWIN)]
            ppv = pv[0] * pv[0]
            phv = pv[0] * he[0]
            ptv = pv[0] * te[0]
            for w in range(1, NWIN):
                ppv = ppv + pv[w] * pv[w]
                phv = phv + pv[w] * he[w]
                ptv = ptv + pv[w] * te[w]
            pp = jnp.sum(ppv)
            ph = jnp.sum(phv)
            pt = jnp.sum(ptv)
            pp_s = jnp.zeros((16,), jnp.float32) + pp
            m = jnp.maximum(_sqrt(pp_s), jnp.float32(1e-12))
            cv = (jnp.zeros((16,), jnp.float32) + (ph - pt)) / (m * m)
            ssv = zero16
            for w in range(NWIN):
                v = he[w] + re[w] - te[w] - cv * pv[w]
                ssv = ssv + v * v
            ss = jnp.sum(ssv)
            return jnp.where(iota == i, jnp.zeros((16,), jnp.float32) + ss,
                             ss_acc)

        ss_acc = lax.fori_loop(0, G, elem_body, zero16)
        scores_v[...] = -_sqrt(ss_acc)
        pltpu.sync_copy(scores_v.at[pl.ds(0, G)],
                        out_hbm.at[pl.ds(base + eb, G)])

    fire(0, bufA, semA)

    def pair_body(c2, _):
        c = c2 * 2
        fire(c + 1, bufB, semB)
        drain(c, bufA, semA)
        compute(c, bufA)

        @pl.when(c + 2 < NCHUNK)
        def _():
            fire(c + 2, bufA, semA)

        drain(c + 1, bufB, semB)
        compute(c + 1, bufB)
        return 0

    lax.fori_loop(0, NCHUNK // 2, pair_body, 0)


_BUF = [
    pltpu.VMEM((G, D), jnp.float32),
    pltpu.VMEM((G, D), jnp.float32),
    pltpu.VMEM((G, DP), jnp.float32),
    pltpu.VMEM((G, DP), jnp.float32),
    pltpu.VMEM((G * L, DP), jnp.float32),
    pltpu.VMEM((G * L, DP), jnp.float32),
    pltpu.VMEM((G * L, DP), jnp.float32),
]

SCRATCH = [
    pltpu.VMEM((EPT,), jnp.int32),
    pltpu.VMEM((EPT * L,), jnp.int32),
    pltpu.VMEM((EPT * L,), jnp.int32),
    pltpu.VMEM((EPT * L,), jnp.int32),
    *_BUF,
    *_BUF,
    pltpu.VMEM((16,), jnp.float32),
    pltpu.SemaphoreType.DMA,
    pltpu.SemaphoreType.DMA,
]


@jax.jit
def _transh_sc(head, relation, tail, hw_flat, rw_flat, tw_flat,
               entity_embedding, rel_emb_p, word_p, proj_p):
    mesh = plsc.VectorSubcoreMesh(core_axis_name="c", subcore_axis_name="s")
    gather_ent = pl.kernel(
        _gath_body,
        out_type=jax.ShapeDtypeStruct((2 * B, D), jnp.float32),
        mesh=mesh,
        compiler_params=pltpu.CompilerParams(
            needs_layout_passes=False, use_tc_tiling_on_sc=True),
        scratch_types=[
            pltpu.VMEM((EPT,), jnp.int32),
            pltpu.VMEM((16, D), jnp.float32),
            pltpu.VMEM((16, D), jnp.float32),
            pltpu.SemaphoreType.DMA,
            pltpu.SemaphoreType.DMA,
        ],
    )
    ent_rows = gather_ent(entity_embedding, head, tail)
    f = pl.kernel(
        _body,
        out_type=jax.ShapeDtypeStruct((B,), jnp.float32),
        mesh=mesh,
        compiler_params=pltpu.CompilerParams(
            needs_layout_passes=False, use_tc_tiling_on_sc=False),
        scratch_types=SCRATCH,
    )
    return f(head, relation, tail, hw_flat, rw_flat, tw_flat,
             ent_rows, rel_emb_p, word_p, proj_p)


def kernel(head, relation, tail, head_w, rel_w, tail_w,
           entity_embedding, relation_embedding, word_embedding,
           relation_projection):
    i32 = jnp.int32
    pad = ((0, 0), (0, DP - D))
    return _transh_sc(
        head.astype(i32), relation.astype(i32), tail.astype(i32),
        head_w.astype(i32).reshape(-1), rel_w.astype(i32).reshape(-1),
        tail_w.astype(i32).reshape(-1),
        entity_embedding,
        jnp.pad(relation_embedding, pad),
        jnp.pad(word_embedding, pad),
        jnp.pad(relation_projection, pad))
